# baseline (speedup 1.0000x reference)
import functools

import jax
import jax.numpy as jnp
from jax import lax
from jax.experimental import pallas as pl
from jax.experimental.pallas import tpu as pltpu
from jax.experimental.pallas import tpu_sc as plsc

NU = 10000
EE = 320000
DIN = 128
HH = 32

NC = 2
NS = 16
LL = 16
G = 128
NG = EE // G
NU_PAD = 10240
ROWS_PER_SUB = NU_PAD // NS
CNT_PAD = 10240
CNT_PER_SUB = CNT_PAD // NS

RB = 1000


def _dense1_body(xu, xr, wul, wur, bu, wrl, wrr, br, p1u, r1u, p1r, r1r):
    xub = xu[...]
    xrb = xr[...]
    p1u[...] = jnp.dot(xub, wul[...], preferred_element_type=jnp.float32)
    r1u[...] = jnp.dot(xub, wur[...], preferred_element_type=jnp.float32) + bu[...]
    p1r[...] = jnp.dot(xrb, wrl[...], preferred_element_type=jnp.float32)
    r1r[...] = jnp.dot(xrb, wrr[...], preferred_element_type=jnp.float32) + br[...]


def _dense2_body(aggr, cntr, r1r, aggu, cntu, r1u, w2ul, w2rl, w2ur, b2u,
                 w2rr, b2r, t2u, t2r, rootr, rootu):
    hr = jax.nn.relu(aggr[...] / jnp.maximum(cntr[...], 1.0) + r1r[...])
    hu = jax.nn.relu(aggu[...] / jnp.maximum(cntu[...], 1.0) + r1u[...])
    t2u[...] = jnp.dot(hu, w2ul[...], preferred_element_type=jnp.float32)
    t2r[...] = jnp.dot(hr, w2rl[...], preferred_element_type=jnp.float32)
    rootr[...] = jnp.dot(hr, w2ur[...], preferred_element_type=jnp.float32) + b2u[...]
    rootu[...] = jnp.dot(hu, w2rr[...], preferred_element_type=jnp.float32) + b2r[...]


def _dense3_body(aggr, cntr, rootr, aggu, cntu, rootu, wdu, bdu, wdr, bdr,
                 uo, ro):
    zr = aggr[...] / jnp.maximum(cntr[...], 1.0) + rootr[...]
    zu = aggu[...] / jnp.maximum(cntu[...], 1.0) + rootu[...]
    uo[...] = jnp.dot(zu, wdu[...], preferred_element_type=jnp.float32) + bdu[...]
    ro[...] = jnp.dot(zr, wdr[...], preferred_element_type=jnp.float32) + bdr[...]


def _row_spec(i):
    return (i, 0)


def _full_spec(i):
    return (0, 0)


def _tc_dense1(xu, xr, wul, wur, bu, wrl, wrr, br):
    grid = (NU // RB,)
    outs = [jax.ShapeDtypeStruct((NU, HH), jnp.float32)] * 4
    return pl.pallas_call(
        _dense1_body,
        grid=grid,
        in_specs=[
            pl.BlockSpec((RB, DIN), _row_spec),
            pl.BlockSpec((RB, DIN), _row_spec),
            pl.BlockSpec((DIN, HH), _full_spec),
            pl.BlockSpec((DIN, HH), _full_spec),
            pl.BlockSpec((1, HH), _full_spec),
            pl.BlockSpec((DIN, HH), _full_spec),
            pl.BlockSpec((DIN, HH), _full_spec),
            pl.BlockSpec((1, HH), _full_spec),
        ],
        out_specs=[pl.BlockSpec((RB, HH), _row_spec)] * 4,
        out_shape=outs,
    )(xu, xr, wul, wur, bu, wrl, wrr, br)


def _tc_dense2(aggr, cntr, r1r, aggu, cntu, r1u, w2ul, w2rl, w2ur, b2u, w2rr, b2r):
    grid = (NU // RB,)
    outs = [jax.ShapeDtypeStruct((NU, HH), jnp.float32)] * 4
    return pl.pallas_call(
        _dense2_body,
        grid=grid,
        in_specs=[
            pl.BlockSpec((RB, HH), _row_spec),
            pl.BlockSpec((RB, 1), _row_spec),
            pl.BlockSpec((RB, HH), _row_spec),
            pl.BlockSpec((RB, HH), _row_spec),
            pl.BlockSpec((RB, 1), _row_spec),
            pl.BlockSpec((RB, HH), _row_spec),
            pl.BlockSpec((HH, HH), _full_spec),
            pl.BlockSpec((HH, HH), _full_spec),
            pl.BlockSpec((HH, HH), _full_spec),
            pl.BlockSpec((1, HH), _full_spec),
            pl.BlockSpec((HH, HH), _full_spec),
            pl.BlockSpec((1, HH), _full_spec),
        ],
        out_specs=[pl.BlockSpec((RB, HH), _row_spec)] * 4,
        out_shape=outs,
    )(aggr, cntr, r1r, aggu, cntu, r1u, w2ul, w2rl, w2ur, b2u, w2rr, b2r)


def _tc_dense3(aggr, cntr, rootr, aggu, cntu, rootu, wdu, bdu, wdr, bdr):
    grid = (NU // RB,)
    outs = [jax.ShapeDtypeStruct((NU, HH), jnp.float32)] * 2
    return pl.pallas_call(
        _dense3_body,
        grid=grid,
        in_specs=[
            pl.BlockSpec((RB, HH), _row_spec),
            pl.BlockSpec((RB, 1), _row_spec),
            pl.BlockSpec((RB, HH), _row_spec),
            pl.BlockSpec((RB, HH), _row_spec),
            pl.BlockSpec((RB, 1), _row_spec),
            pl.BlockSpec((RB, HH), _row_spec),
            pl.BlockSpec((HH, HH), _full_spec),
            pl.BlockSpec((1, HH), _full_spec),
            pl.BlockSpec((HH, HH), _full_spec),
            pl.BlockSpec((1, HH), _full_spec),
        ],
        out_specs=[pl.BlockSpec((RB, HH), _row_spec)] * 2,
        out_shape=outs,
    )(aggr, cntr, rootr, aggu, cntu, rootu, wdu, bdu, wdr, bdr)


NG_PAD = 2560
E_PAD = NG_PAD * G


def _zero_vmem_2d(ref, nrows):
    z16 = jnp.zeros((LL,), jnp.float32)

    def zr(i, _):
        ref[i, pl.ds(0, LL)] = z16
        ref[i, pl.ds(LL, LL)] = z16
        return 0

    lax.fori_loop(0, nrows, zr, 0)


def _fill_vmem_1d(ref, n, val):
    v16 = jnp.full((LL,), val, jnp.float32)

    def zr(i, _):
        ref[pl.ds(i * LL, LL)] = v16
        return 0

    lax.fori_loop(0, n // LL, zr, 0)


KPIPE = 10
GPT = NG_PAD // NS
ZROWS = 160


def _agg_body(with_counts, *refs):
    if with_counts:
        (tabu, tabr, src2, dst2, agg_r, agg_u, cnt_r_o, cnt_u_o,
         gidx, sidx, zrow, zcnt, ones, acc, cntp, *rest) = refs
    else:
        (tabu, tabr, src2, dst2, agg_r, agg_u,
         gidx, sidx, zrow, acc, *rest) = refs
    rows = rest[:KPIPE]
    gsems = rest[KPIPE:2 * KPIPE]
    ssems = rest[2 * KPIPE:3 * KPIPE]
    c = lax.axis_index("c")
    s = lax.axis_index("s")

    _zero_vmem_2d(zrow, ZROWS)
    for q in range(ROWS_PER_SUB // ZROWS):
        pltpu.sync_copy(zrow, acc.at[pl.ds(s * ROWS_PER_SUB + q * ZROWS, ZROWS)])
    if with_counts:
        _fill_vmem_1d(zcnt, CNT_PER_SUB, 0.0)
        _fill_vmem_1d(ones, G, 1.0)
        pltpu.sync_copy(zcnt, cntp.at[pl.ds(s * CNT_PER_SUB, CNT_PER_SUB)])
    plsc.subcore_barrier()

    base = s * GPT

    def direction(tab, gsel2, ssel2):
        pltpu.sync_copy(gsel2.at[pl.ds(base, GPT)], gidx)
        pltpu.sync_copy(ssel2.at[pl.ds(base, GPT)], sidx)
        for b in range(KPIPE):
            pltpu.async_copy(tab.at[gidx.at[b]], rows[b], gsems[b])

        def it(i, _):
            for b in range(KPIPE):
                g = i * KPIPE + b

                @pl.when(base + g < NG)
                def _():
                    pltpu.make_async_copy(tab.at[gidx.at[g]], rows[b],
                                          gsems[b]).wait()
                    pltpu.async_copy(rows[b], acc.at[sidx.at[g]], ssems[b],
                                     add=True)
                    if with_counts:
                        pltpu.async_copy(ones, cntp.at[sidx.at[g]], ssems[b],
                                         add=True)
            for b in range(KPIPE):
                g = i * KPIPE + b
                gn = g + KPIPE

                @pl.when((gn < GPT) & (base + gn < NG))
                def _():
                    pltpu.make_async_copy(rows[b], acc.at[sidx.at[g]],
                                          ssems[b]).wait()
                    if with_counts:
                        pltpu.make_async_copy(ones, cntp.at[sidx.at[g]],
                                              ssems[b]).wait()
                    pltpu.async_copy(tab.at[gidx.at[gn]], rows[b], gsems[b])
            return 0

        lax.fori_loop(0, GPT // KPIPE, it, 0)
        for b in range(KPIPE):
            pltpu.make_async_copy(rows[b], acc.at[sidx.at[b]], ssems[b]).wait()
            if with_counts:
                pltpu.make_async_copy(ones, cntp.at[sidx.at[b]],
                                      ssems[b]).wait()

    @pl.when(c == 0)
    def _():
        direction(tabu, src2, dst2)

    @pl.when(c == 1)
    def _():
        direction(tabr, dst2, src2)

    plsc.subcore_barrier()
    rs = pl.ds(s * ROWS_PER_SUB, ROWS_PER_SUB)
    cs = pl.ds(s * CNT_PER_SUB, CNT_PER_SUB)

    @pl.when(c == 0)
    def _():
        pltpu.sync_copy(acc.at[rs], agg_r.at[rs])
        if with_counts:
            pltpu.sync_copy(cntp.at[cs], cnt_r_o.at[cs])

    @pl.when(c == 1)
    def _():
        pltpu.sync_copy(acc.at[rs], agg_u.at[rs])
        if with_counts:
            pltpu.sync_copy(cntp.at[cs], cnt_u_o.at[cs])


def _make_agg(with_counts):
    outs = [jax.ShapeDtypeStruct((NU_PAD, HH), jnp.float32)] * 2
    scratch = [
        pltpu.VMEM((GPT, G), jnp.int32),
        pltpu.VMEM((GPT, G), jnp.int32),
        pltpu.VMEM((ZROWS, HH), jnp.float32),
    ]
    if with_counts:
        outs += [jax.ShapeDtypeStruct((CNT_PAD,), jnp.float32)] * 2
        scratch += [
            pltpu.VMEM((CNT_PER_SUB,), jnp.float32),
            pltpu.VMEM((G,), jnp.float32),
        ]
    scratch += [pltpu.VMEM_SHARED((NU_PAD, HH), jnp.float32)]
    if with_counts:
        scratch += [pltpu.VMEM_SHARED((CNT_PAD,), jnp.float32)]
    scratch += [pltpu.VMEM((G, HH), jnp.float32)] * KPIPE
    scratch += [pltpu.SemaphoreType.DMA] * (2 * KPIPE)
    mesh = plsc.VectorSubcoreMesh(core_axis_name="c", subcore_axis_name="s")
    return pl.kernel(
        functools.partial(_agg_body, with_counts),
        out_type=outs,
        mesh=mesh,
        scratch_types=scratch,
        compiler_params=pltpu.CompilerParams(use_tc_tiling_on_sc=False),
    )


GPT_D = NG_PAD // (NC * NS)


GPT_D = NG_PAD // (NC * NS)
KDEC = 8


def _dot_group(ub, rb, sbuf):
    lanes = lax.iota(jnp.int32, LL)

    def blk(i, _):
        acc = jnp.zeros((LL,), jnp.float32)
        for q in range(LL):
            e = i * LL + q
            v = (ub[e, pl.ds(0, LL)] * rb[e, pl.ds(0, LL)]
                 + ub[e, pl.ds(LL, LL)] * rb[e, pl.ds(LL, LL)])
            acc = jnp.where(lanes == q, jnp.sum(v), acc)
        sbuf[pl.ds(i * LL, LL)] = acc
        return 0

    lax.fori_loop(0, G // LL, blk, 0)


def _decode_body(utab, rtab, row2, col2, out_s, *rest):
    ridx = rest[0]
    cidx = rest[1]
    ub = rest[2:2 + KDEC]
    rb = rest[2 + KDEC:2 + 2 * KDEC]
    sbufs = rest[2 + 2 * KDEC:2 + 3 * KDEC]
    off0 = 2 + 3 * KDEC
    gusems = rest[off0:off0 + KDEC]
    grsems = rest[off0 + KDEC:off0 + 2 * KDEC]
    wsems = rest[off0 + 2 * KDEC:off0 + 3 * KDEC]
    c = lax.axis_index("c")
    s = lax.axis_index("s")
    wid = s * NC + c
    base = wid * GPT_D

    pltpu.sync_copy(row2.at[pl.ds(base, GPT_D)], ridx)
    pltpu.sync_copy(col2.at[pl.ds(base, GPT_D)], cidx)
    for b in range(KDEC):
        pltpu.async_copy(utab.at[ridx.at[b]], ub[b], gusems[b])
        pltpu.async_copy(rtab.at[cidx.at[b]], rb[b], grsems[b])

    def it(i, _):
        for b in range(KDEC):
            g = i * KDEC + b

            @pl.when(base + g < NG)
            def _():
                pltpu.make_async_copy(utab.at[ridx.at[g]], ub[b],
                                      gusems[b]).wait()
                pltpu.make_async_copy(rtab.at[cidx.at[g]], rb[b],
                                      grsems[b]).wait()
                _dot_group(ub[b], rb[b], sbufs[b])
                pltpu.async_copy(sbufs[b], out_s.at[pl.ds((base + g) * G, G)],
                                 wsems[b])
        for b in range(KDEC):
            g = i * KDEC + b
            gn = g + KDEC

            @pl.when((gn < GPT_D) & (base + gn < NG))
            def _():
                pltpu.make_async_copy(sbufs[b],
                                      out_s.at[pl.ds((base + g) * G, G)],
                                      wsems[b]).wait()
                pltpu.async_copy(utab.at[ridx.at[gn]], ub[b], gusems[b])
                pltpu.async_copy(rtab.at[cidx.at[gn]], rb[b], grsems[b])
        return 0

    lax.fori_loop(0, GPT_D // KDEC, it, 0)
    for b in range(KDEC):
        pltpu.make_async_copy(sbufs[b], out_s.at[pl.ds(base * G, G)],
                              wsems[b]).wait()


def _make_decode():
    outs = jax.ShapeDtypeStruct((E_PAD,), jnp.float32)
    scratch = [
        pltpu.VMEM((GPT_D, G), jnp.int32),
        pltpu.VMEM((GPT_D, G), jnp.int32),
    ]
    scratch += [pltpu.VMEM((G, HH), jnp.float32)] * (2 * KDEC)
    scratch += [pltpu.VMEM((G,), jnp.float32)] * KDEC
    scratch += [pltpu.SemaphoreType.DMA] * (3 * KDEC)
    mesh = plsc.VectorSubcoreMesh(core_axis_name="c", subcore_axis_name="s")
    return pl.kernel(_decode_body, out_type=outs, mesh=mesh,
                     scratch_types=scratch,
                     compiler_params=pltpu.CompilerParams(
                         use_tc_tiling_on_sc=False,
                         needs_layout_passes=False))


def kernel(x_user, x_restaurant, edge_index, edge_label_index,
           W1_u2r_l, W1_u2r_r, b1_u2r, W1_r2u_l, W1_r2u_r, b1_r2u,
           W2_u2r_l, W2_u2r_r, b2_u2r, W2_r2u_l, W2_r2u_r, b2_r2u,
           Wd_user, bd_user, Wd_rest, bd_rest):
    padz = jnp.zeros((1, E_PAD - EE), jnp.int32)
    ei = jnp.concatenate([edge_index, jnp.broadcast_to(padz, (2, E_PAD - EE))],
                         axis=1)
    eli = jnp.concatenate([edge_label_index,
                           jnp.broadcast_to(padz, (2, E_PAD - EE))], axis=1)
    src = ei[0].reshape(NG_PAD, G)
    dst = ei[1].reshape(NG_PAD, G)
    row = eli[0].reshape(NG_PAD, G)
    col = eli[1].reshape(NG_PAD, G)

    p1u, r1u, p1r, r1r = _tc_dense1(
        x_user, x_restaurant,
        W1_u2r_l, W1_r2u_r, b1_r2u.reshape(1, HH),
        W1_r2u_l, W1_u2r_r, b1_u2r.reshape(1, HH))

    agg_r, agg_u, cnt_r_p, cnt_u_p = _make_agg(True)(p1u, p1r, src, dst)
    cnt_r = cnt_r_p.reshape(CNT_PAD, 1)
    cnt_u = cnt_u_p.reshape(CNT_PAD, 1)

    t2u, t2r, root_r, root_u = _tc_dense2(
        agg_r, cnt_r, r1r, agg_u, cnt_u, r1u,
        W2_u2r_l, W2_r2u_l, W2_u2r_r, b2_u2r.reshape(1, HH),
        W2_r2u_r, b2_r2u.reshape(1, HH))

    agg2_r, agg2_u = _make_agg(False)(t2u, t2r, src, dst)

    uemb, remb = _tc_dense3(
        agg2_r, cnt_r, root_r, agg2_u, cnt_u, root_u,
        Wd_user, bd_user.reshape(1, HH), Wd_rest, bd_rest.reshape(1, HH))

    score = _make_decode()(uemb, remb, row, col)
    return score[:EE]


# speedup vs baseline: 1.0538x; 1.0538x over previous
import functools

import jax
import jax.numpy as jnp
from jax import lax
from jax.experimental import pallas as pl
from jax.experimental.pallas import tpu as pltpu
from jax.experimental.pallas import tpu_sc as plsc

NU = 10000
EE = 320000
DIN = 128
HH = 32

NC = 2
NS = 16
LL = 16
G = 128
NG = EE // G
NU_PAD = 10240
ROWS_PER_SUB = NU_PAD // NS
CNT_PAD = 10240
CNT_PER_SUB = CNT_PAD // NS

RB = 1000


def _dense1_body(xu, xr, wul, wur, bu, wrl, wrr, br, p1u, r1u, p1r, r1r):
    xub = xu[...]
    xrb = xr[...]
    p1u[...] = jnp.dot(xub, wul[...], preferred_element_type=jnp.float32)
    r1u[...] = jnp.dot(xub, wur[...], preferred_element_type=jnp.float32) + bu[...]
    p1r[...] = jnp.dot(xrb, wrl[...], preferred_element_type=jnp.float32)
    r1r[...] = jnp.dot(xrb, wrr[...], preferred_element_type=jnp.float32) + br[...]


def _dense2_body(aggr, cntr, r1r, aggu, cntu, r1u, w2ul, w2rl, w2ur, b2u,
                 w2rr, b2r, t2u, t2r, rootr, rootu):
    ar = aggr[...][:, :HH]
    au = aggu[...][:, :HH]
    hr = jax.nn.relu(ar / jnp.maximum(cntr[...], 1.0) + r1r[...])
    hu = jax.nn.relu(au / jnp.maximum(cntu[...], 1.0) + r1u[...])
    t2u[...] = jnp.dot(hu, w2ul[...], preferred_element_type=jnp.float32)
    t2r[...] = jnp.dot(hr, w2rl[...], preferred_element_type=jnp.float32)
    rootr[...] = jnp.dot(hr, w2ur[...], preferred_element_type=jnp.float32) + b2u[...]
    rootu[...] = jnp.dot(hu, w2rr[...], preferred_element_type=jnp.float32) + b2r[...]


def _dense3_body(aggr, cntr, rootr, aggu, cntu, rootu, wdu, bdu, wdr, bdr,
                 uo, ro):
    zr = aggr[...][:, :HH] / jnp.maximum(cntr[...], 1.0) + rootr[...]
    zu = aggu[...][:, :HH] / jnp.maximum(cntu[...], 1.0) + rootu[...]
    uo[...] = jnp.dot(zu, wdu[...], preferred_element_type=jnp.float32) + bdu[...]
    ro[...] = jnp.dot(zr, wdr[...], preferred_element_type=jnp.float32) + bdr[...]


def _row_spec(i):
    return (i, 0)


def _full_spec(i):
    return (0, 0)


def _tc_dense1(xu, xr, wul, wur, bu, wrl, wrr, br):
    grid = (NU // RB,)
    outs = [jax.ShapeDtypeStruct((NU, HH), jnp.float32)] * 4
    return pl.pallas_call(
        _dense1_body,
        grid=grid,
        in_specs=[
            pl.BlockSpec((RB, DIN), _row_spec),
            pl.BlockSpec((RB, DIN), _row_spec),
            pl.BlockSpec((DIN, HH), _full_spec),
            pl.BlockSpec((DIN, HH), _full_spec),
            pl.BlockSpec((1, HH), _full_spec),
            pl.BlockSpec((DIN, HH), _full_spec),
            pl.BlockSpec((DIN, HH), _full_spec),
            pl.BlockSpec((1, HH), _full_spec),
        ],
        out_specs=[pl.BlockSpec((RB, HH), _row_spec)] * 4,
        out_shape=outs,
    )(xu, xr, wul, wur, bu, wrl, wrr, br)


def _tc_dense2(aggr, cntr, r1r, aggu, cntu, r1u, w2ul, w2rl, w2ur, b2u, w2rr, b2r):
    grid = (NU // RB,)
    outs = [jax.ShapeDtypeStruct((NU, HH), jnp.float32)] * 4
    return pl.pallas_call(
        _dense2_body,
        grid=grid,
        in_specs=[
            pl.BlockSpec((RB, DIN), _row_spec),
            pl.BlockSpec((RB, 1), _row_spec),
            pl.BlockSpec((RB, HH), _row_spec),
            pl.BlockSpec((RB, DIN), _row_spec),
            pl.BlockSpec((RB, 1), _row_spec),
            pl.BlockSpec((RB, HH), _row_spec),
            pl.BlockSpec((HH, HH), _full_spec),
            pl.BlockSpec((HH, HH), _full_spec),
            pl.BlockSpec((HH, HH), _full_spec),
            pl.BlockSpec((1, HH), _full_spec),
            pl.BlockSpec((HH, HH), _full_spec),
            pl.BlockSpec((1, HH), _full_spec),
        ],
        out_specs=[pl.BlockSpec((RB, HH), _row_spec)] * 4,
        out_shape=outs,
    )(aggr, cntr, r1r, aggu, cntu, r1u, w2ul, w2rl, w2ur, b2u, w2rr, b2r)


def _tc_dense3(aggr, cntr, rootr, aggu, cntu, rootu, wdu, bdu, wdr, bdr):
    grid = (NU // RB,)
    outs = [jax.ShapeDtypeStruct((NU, HH), jnp.float32)] * 2
    return pl.pallas_call(
        _dense3_body,
        grid=grid,
        in_specs=[
            pl.BlockSpec((RB, DIN), _row_spec),
            pl.BlockSpec((RB, 1), _row_spec),
            pl.BlockSpec((RB, HH), _row_spec),
            pl.BlockSpec((RB, DIN), _row_spec),
            pl.BlockSpec((RB, 1), _row_spec),
            pl.BlockSpec((RB, HH), _row_spec),
            pl.BlockSpec((HH, HH), _full_spec),
            pl.BlockSpec((1, HH), _full_spec),
            pl.BlockSpec((HH, HH), _full_spec),
            pl.BlockSpec((1, HH), _full_spec),
        ],
        out_specs=[pl.BlockSpec((RB, HH), _row_spec)] * 2,
        out_shape=outs,
    )(aggr, cntr, rootr, aggu, cntu, rootu, wdu, bdu, wdr, bdr)


NG_PAD = 2560
E_PAD = NG_PAD * G


def _zero_vmem_2d(ref, nrows):
    z16 = jnp.zeros((LL,), jnp.float32)

    def zr(i, _):
        ref[i, pl.ds(0, LL)] = z16
        ref[i, pl.ds(LL, LL)] = z16
        return 0

    lax.fori_loop(0, nrows, zr, 0)


def _fill_vmem_1d(ref, n, val):
    v16 = jnp.full((LL,), val, jnp.float32)

    def zr(i, _):
        ref[pl.ds(i * LL, LL)] = v16
        return 0

    lax.fori_loop(0, n // LL, zr, 0)


KPIPE = 8
GPT = NG_PAD // NS
ZROWS = 160


def _agg_body(with_counts, *refs):
    if with_counts:
        (tabu, tabr, src2, dst2, agg_r, agg_u, cnt_r_o, cnt_u_o,
         gidx, sidx, zrow, zcnt, ones, acc, cntp, *rest) = refs
    else:
        (tabu, tabr, src2, dst2, agg_r, agg_u,
         gidx, sidx, zrow, acc, *rest) = refs
    rows = rest[:KPIPE]
    gsems = rest[KPIPE:2 * KPIPE]
    ssems = rest[2 * KPIPE:3 * KPIPE]
    c = lax.axis_index("c")
    s = lax.axis_index("s")

    _zero_vmem_2d(zrow, ZROWS)
    for q in range(ROWS_PER_SUB // ZROWS):
        pltpu.sync_copy(zrow, acc.at[pl.ds(s * ROWS_PER_SUB + q * ZROWS, ZROWS)])
    if with_counts:
        _fill_vmem_1d(zcnt, CNT_PER_SUB, 0.0)
        _fill_vmem_1d(ones, G, 1.0)
        pltpu.sync_copy(zcnt, cntp.at[pl.ds(s * CNT_PER_SUB, CNT_PER_SUB)])
    plsc.subcore_barrier()

    base = s * GPT

    def direction(tab, gsel2, ssel2):
        pltpu.sync_copy(gsel2.at[pl.ds(base, GPT)], gidx)
        pltpu.sync_copy(ssel2.at[pl.ds(base, GPT)], sidx)
        for b in range(KPIPE):
            pltpu.async_copy(tab.at[gidx.at[b]], rows[b], gsems[b])

        def it(i, _):
            for b in range(KPIPE):
                g = i * KPIPE + b

                @pl.when(base + g < NG)
                def _():
                    pltpu.make_async_copy(tab.at[gidx.at[g]], rows[b],
                                          gsems[b]).wait()
                    pltpu.async_copy(rows[b], acc.at[sidx.at[g]], ssems[b],
                                     add=True)
                    if with_counts:
                        pltpu.async_copy(ones, cntp.at[sidx.at[g]], ssems[b],
                                         add=True)
            for b in range(KPIPE):
                g = i * KPIPE + b
                gn = g + KPIPE

                @pl.when((gn < GPT) & (base + gn < NG))
                def _():
                    pltpu.make_async_copy(rows[b], acc.at[sidx.at[g]],
                                          ssems[b]).wait()
                    if with_counts:
                        pltpu.make_async_copy(ones, cntp.at[sidx.at[g]],
                                              ssems[b]).wait()
                    pltpu.async_copy(tab.at[gidx.at[gn]], rows[b], gsems[b])
            return 0

        lax.fori_loop(0, GPT // KPIPE, it, 0)
        for b in range(KPIPE):
            pltpu.make_async_copy(rows[b], acc.at[sidx.at[b]], ssems[b]).wait()
            if with_counts:
                pltpu.make_async_copy(ones, cntp.at[sidx.at[b]],
                                      ssems[b]).wait()

    @pl.when(c == 0)
    def _():
        direction(tabu, src2, dst2)

    @pl.when(c == 1)
    def _():
        direction(tabr, dst2, src2)

    plsc.subcore_barrier()
    cs = pl.ds(s * CNT_PER_SUB, CNT_PER_SUB)

    def writeback(out):
        for q in range(ROWS_PER_SUB // ZROWS):
            r0 = s * ROWS_PER_SUB + q * ZROWS
            pltpu.sync_copy(acc.at[pl.ds(r0, ZROWS)], zrow)
            pltpu.sync_copy(zrow, out.at[pl.ds(r0, ZROWS), pl.ds(0, HH)])

    @pl.when(c == 0)
    def _():
        writeback(agg_r)
        if with_counts:
            pltpu.sync_copy(cntp.at[cs], cnt_r_o.at[cs])

    @pl.when(c == 1)
    def _():
        writeback(agg_u)
        if with_counts:
            pltpu.sync_copy(cntp.at[cs], cnt_u_o.at[cs])


def _make_agg(with_counts):
    outs = [jax.ShapeDtypeStruct((NU_PAD, DIN), jnp.float32)] * 2
    scratch = [
        pltpu.VMEM((GPT, G), jnp.int32),
        pltpu.VMEM((GPT, G), jnp.int32),
        pltpu.VMEM((ZROWS, HH), jnp.float32),
    ]
    if with_counts:
        outs += [jax.ShapeDtypeStruct((CNT_PAD,), jnp.float32)] * 2
        scratch += [
            pltpu.VMEM((CNT_PER_SUB,), jnp.float32),
            pltpu.VMEM((G,), jnp.float32),
        ]
    scratch += [pltpu.VMEM_SHARED((NU_PAD, HH), jnp.float32)]
    if with_counts:
        scratch += [pltpu.VMEM_SHARED((CNT_PAD,), jnp.float32)]
    scratch += [pltpu.VMEM((G, HH), jnp.float32)] * KPIPE
    scratch += [pltpu.SemaphoreType.DMA] * (2 * KPIPE)
    mesh = plsc.VectorSubcoreMesh(core_axis_name="c", subcore_axis_name="s")
    return pl.kernel(
        functools.partial(_agg_body, with_counts),
        out_type=outs,
        mesh=mesh,
        scratch_types=scratch,
        compiler_params=pltpu.CompilerParams(use_tc_tiling_on_sc=False),
    )


GPT_D = NG_PAD // (NC * NS)


GPT_D = NG_PAD // (NC * NS)
KDEC = 8


def _dot_group(ub, rb, sbuf):
    lanes = lax.iota(jnp.int32, LL)

    def blk(i, _):
        acc = jnp.zeros((LL,), jnp.float32)
        for q in range(LL):
            e = i * LL + q
            v = (ub[e, pl.ds(0, LL)] * rb[e, pl.ds(0, LL)]
                 + ub[e, pl.ds(LL, LL)] * rb[e, pl.ds(LL, LL)])
            acc = jnp.where(lanes == q, jnp.sum(v), acc)
        sbuf[pl.ds(i * LL, LL)] = acc
        return 0

    lax.fori_loop(0, G // LL, blk, 0)


def _decode_body(utab, rtab, row2, col2, out_s, *rest):
    ridx = rest[0]
    cidx = rest[1]
    ub = rest[2:2 + KDEC]
    rb = rest[2 + KDEC:2 + 2 * KDEC]
    sbufs = rest[2 + 2 * KDEC:2 + 3 * KDEC]
    off0 = 2 + 3 * KDEC
    gusems = rest[off0:off0 + KDEC]
    grsems = rest[off0 + KDEC:off0 + 2 * KDEC]
    wsems = rest[off0 + 2 * KDEC:off0 + 3 * KDEC]
    c = lax.axis_index("c")
    s = lax.axis_index("s")
    wid = s * NC + c
    base = wid * GPT_D

    pltpu.sync_copy(row2.at[pl.ds(base, GPT_D)], ridx)
    pltpu.sync_copy(col2.at[pl.ds(base, GPT_D)], cidx)
    for b in range(KDEC):
        pltpu.async_copy(utab.at[ridx.at[b]], ub[b], gusems[b])
        pltpu.async_copy(rtab.at[cidx.at[b]], rb[b], grsems[b])

    def it(i, _):
        for b in range(KDEC):
            g = i * KDEC + b

            @pl.when(base + g < NG)
            def _():
                pltpu.make_async_copy(utab.at[ridx.at[g]], ub[b],
                                      gusems[b]).wait()
                pltpu.make_async_copy(rtab.at[cidx.at[g]], rb[b],
                                      grsems[b]).wait()
                _dot_group(ub[b], rb[b], sbufs[b])
                pltpu.async_copy(sbufs[b], out_s.at[pl.ds((base + g) * G, G)],
                                 wsems[b])
        for b in range(KDEC):
            g = i * KDEC + b
            gn = g + KDEC

            @pl.when((gn < GPT_D) & (base + gn < NG))
            def _():
                pltpu.make_async_copy(sbufs[b],
                                      out_s.at[pl.ds((base + g) * G, G)],
                                      wsems[b]).wait()
                pltpu.async_copy(utab.at[ridx.at[gn]], ub[b], gusems[b])
                pltpu.async_copy(rtab.at[cidx.at[gn]], rb[b], grsems[b])
        return 0

    lax.fori_loop(0, GPT_D // KDEC, it, 0)
    for b in range(KDEC):
        pltpu.make_async_copy(sbufs[b], out_s.at[pl.ds(base * G, G)],
                              wsems[b]).wait()


def _make_decode():
    outs = jax.ShapeDtypeStruct((E_PAD,), jnp.float32)
    scratch = [
        pltpu.VMEM((GPT_D, G), jnp.int32),
        pltpu.VMEM((GPT_D, G), jnp.int32),
    ]
    scratch += [pltpu.VMEM((G, HH), jnp.float32)] * (2 * KDEC)
    scratch += [pltpu.VMEM((G,), jnp.float32)] * KDEC
    scratch += [pltpu.SemaphoreType.DMA] * (3 * KDEC)
    mesh = plsc.VectorSubcoreMesh(core_axis_name="c", subcore_axis_name="s")
    return pl.kernel(_decode_body, out_type=outs, mesh=mesh,
                     scratch_types=scratch,
                     compiler_params=pltpu.CompilerParams(
                         use_tc_tiling_on_sc=False,
                         needs_layout_passes=False))


def kernel(x_user, x_restaurant, edge_index, edge_label_index,
           W1_u2r_l, W1_u2r_r, b1_u2r, W1_r2u_l, W1_r2u_r, b1_r2u,
           W2_u2r_l, W2_u2r_r, b2_u2r, W2_r2u_l, W2_r2u_r, b2_r2u,
           Wd_user, bd_user, Wd_rest, bd_rest):
    padz = jnp.zeros((1, E_PAD - EE), jnp.int32)
    ei = jnp.concatenate([edge_index, jnp.broadcast_to(padz, (2, E_PAD - EE))],
                         axis=1)
    eli = jnp.concatenate([edge_label_index,
                           jnp.broadcast_to(padz, (2, E_PAD - EE))], axis=1)
    src = ei[0].reshape(NG_PAD, G)
    dst = ei[1].reshape(NG_PAD, G)
    row = eli[0].reshape(NG_PAD, G)
    col = eli[1].reshape(NG_PAD, G)

    p1u, r1u, p1r, r1r = _tc_dense1(
        x_user, x_restaurant,
        W1_u2r_l, W1_r2u_r, b1_r2u.reshape(1, HH),
        W1_r2u_l, W1_u2r_r, b1_u2r.reshape(1, HH))

    agg_r, agg_u, cnt_r_p, cnt_u_p = _make_agg(True)(p1u, p1r, src, dst)
    cnt_r = cnt_r_p.reshape(CNT_PAD, 1)
    cnt_u = cnt_u_p.reshape(CNT_PAD, 1)

    t2u, t2r, root_r, root_u = _tc_dense2(
        agg_r, cnt_r, r1r, agg_u, cnt_u, r1u,
        W2_u2r_l, W2_r2u_l, W2_u2r_r, b2_u2r.reshape(1, HH),
        W2_r2u_r, b2_r2u.reshape(1, HH))

    agg2_r, agg2_u = _make_agg(False)(t2u, t2r, src, dst)

    uemb, remb = _tc_dense3(
        agg2_r, cnt_r, root_r, agg2_u, cnt_u, root_u,
        Wd_user, bd_user.reshape(1, HH), Wd_rest, bd_rest.reshape(1, HH))

    score = _make_decode()(uemb, remb, row, col)
    return score[:EE]
